# baseline (speedup 1.0000x reference)
import functools
import jax
import jax.numpy as jnp
from jax import lax
from jax.experimental import pallas as pl
from jax.experimental.pallas import tpu as pltpu
from jax.experimental.pallas import tpu_sc as plsc

_N = 10000
_E = 320000
_CH = 128
_H = 64
_G = 128
_CLS = 10

_NC = 2
_NS = 16
_NW = _NC * _NS
_L = 16

_DEG_EPT = _E // _NW
_DEG_ROWS = _N // _L

_CHUNK = 128
_NCHUNK = 80
_EPT = _CHUNK * _NCHUNK
_EPAD = _EPT * _NW
_ZROWS = 125
_SROWS = _N // _NS

_mesh = plsc.VectorSubcoreMesh(core_axis_name="c", subcore_axis_name="s")


@functools.partial(
    pl.kernel,
    out_type=jax.ShapeDtypeStruct((_NW, _N), jnp.float32),
    mesh=_mesh,
    compiler_params=pltpu.CompilerParams(needs_layout_passes=False, use_tc_tiling_on_sc=False),
    scratch_types=[
        pltpu.VMEM((_DEG_EPT,), jnp.int32),
        pltpu.VMEM((_DEG_EPT,), jnp.float32),
        pltpu.VMEM((_N,), jnp.float32),
    ],
)
def _deg_kernel(dst_hbm, ew_hbm, out_hbm, dstv, ewv, degl):
    cid = lax.axis_index("c")
    sid = lax.axis_index("s")
    wid = sid * _NC + cid
    base = wid * _DEG_EPT
    pltpu.sync_copy(dst_hbm.at[pl.ds(base, _DEG_EPT)], dstv)
    pltpu.sync_copy(ew_hbm.at[pl.ds(base, _DEG_EPT)], ewv)

    def zero_body(i, carry):
        degl[pl.ds(i * _L, _L)] = jnp.zeros((_L,), jnp.float32)
        return carry

    lax.fori_loop(0, _N // _L, zero_body, 0)

    def edge_body(i, carry):
        d16 = dstv[pl.ds(i * _L, _L)]
        w16 = ewv[pl.ds(i * _L, _L)]
        plsc.addupdate_scatter(degl, [d16], w16)
        return carry

    lax.fori_loop(0, _DEG_EPT // _L, edge_body, 0)
    pltpu.sync_copy(degl, out_hbm.at[wid])


@functools.partial(
    pl.kernel,
    out_type=jax.ShapeDtypeStruct((_NC, _N, _H), jnp.float32),
    mesh=_mesh,
    compiler_params=pltpu.CompilerParams(needs_layout_passes=False, use_tc_tiling_on_sc=False),
    scratch_types=[
        pltpu.VMEM((_NCHUNK, _CHUNK), jnp.int32),
        pltpu.VMEM((_NCHUNK, _CHUNK), jnp.int32),
        pltpu.VMEM((_NCHUNK, _CHUNK), jnp.float32),
        pltpu.VMEM((_CHUNK, _H), jnp.float32),
        pltpu.VMEM((_CHUNK, _H), jnp.float32),
        pltpu.VMEM((_ZROWS, _H), jnp.float32),
        pltpu.VMEM_SHARED((_N, _H), jnp.float32),
        pltpu.SemaphoreType.DMA,
        pltpu.SemaphoreType.DMA,
        pltpu.SemaphoreType.DMA,
        pltpu.SemaphoreType.DMA,
    ],
)
def _agg_kernel(src_hbm, dst_hbm, ew_hbm, hs_hbm, out_hbm,
                src2d, dst2d, ew2d, rows0, rows1, zbuf, acc,
                sg0, sg1, ss0, ss1):
    cid = lax.axis_index("c")
    sid = lax.axis_index("s")
    wid = sid * _NC + cid
    pltpu.sync_copy(src_hbm.at[wid], src2d)
    pltpu.sync_copy(dst_hbm.at[wid], dst2d)
    pltpu.sync_copy(ew_hbm.at[wid], ew2d)

    def zrow_body(i, carry):
        for c4 in range(_H // _L):
            zbuf[i, pl.ds(c4 * _L, _L)] = jnp.zeros((_L,), jnp.float32)
        return carry

    lax.fori_loop(0, _ZROWS, zrow_body, 0)
    for k in range(_SROWS // _ZROWS):
        pltpu.sync_copy(zbuf, acc.at[pl.ds(sid * _SROWS + k * _ZROWS, _ZROWS)])
    plsc.subcore_barrier()

    def scale(rows, ci):
        def group_body(g, gcarry):
            ew16 = ew2d[ci, pl.ds(g * _L, _L)]
            for l in range(_L):
                s = ew16[l]
                e = g * _L + l
                for c4 in range(_H // _L):
                    rows[e, pl.ds(c4 * _L, _L)] = rows[e, pl.ds(c4 * _L, _L)] * s
            return gcarry

        lax.fori_loop(0, _CHUNK // _L, group_body, 0)

    def pair_body(g, carry):
        c0 = 2 * g
        c1 = 2 * g + 1
        dg0 = pltpu.async_copy(hs_hbm.at[src2d.at[c0]], rows0, sg0)
        dg1 = pltpu.async_copy(hs_hbm.at[src2d.at[c1]], rows1, sg1)
        dg0.wait()
        scale(rows0, c0)
        ds0 = pltpu.async_copy(rows0, acc.at[dst2d.at[c0]], ss0, add=True)
        dg1.wait()
        scale(rows1, c1)
        ds1 = pltpu.async_copy(rows1, acc.at[dst2d.at[c1]], ss1, add=True)
        ds0.wait()
        ds1.wait()
        return carry

    lax.fori_loop(0, _NCHUNK // 2, pair_body, 0)
    plsc.subcore_barrier()
    pltpu.sync_copy(acc.at[pl.ds(sid * _SROWS, _SROWS)],
                    out_hbm.at[cid, pl.ds(sid * _SROWS, _SROWS)])


def _prep_body(dp_ref, x_ref, w1_ref, dinv_ref, hs_ref):
    deg = jnp.sum(dp_ref[...], axis=1, keepdims=True) + 1.0
    dinv = lax.rsqrt(deg)
    dinv_ref[...] = dinv
    xw = jnp.dot(x_ref[...], w1_ref[...], preferred_element_type=jnp.float32)
    hs_ref[...] = xw * dinv


def _mid_body(p_ref, hs_ref, dinv_ref, b_ref, w_ref, out_ref):
    dinv = dinv_ref[...]
    h = dinv * (p_ref[0] + p_ref[1] + hs_ref[...]) + b_ref[...]
    h = jnp.maximum(h, 0.0)
    out_ref[...] = jnp.dot(h, w_ref[...], preferred_element_type=jnp.float32) * dinv


def _head_body(p_ref, hs_ref, dinv_ref, b_ref, gamma_ref, beta_ref,
               batch_ref, wm1_ref, bm1_ref, wm2_ref, bm2_ref, out_ref):
    dinv = dinv_ref[...]
    h = dinv * (p_ref[0] + p_ref[1] + hs_ref[...]) + b_ref[...]
    h = jnp.maximum(h, 0.0)
    mean = jnp.mean(h, axis=0, keepdims=True)
    d = h - mean
    var = jnp.mean(d * d, axis=0, keepdims=True)
    hn = d * lax.rsqrt(var + 1e-5) * gamma_ref[...] + beta_ref[...]
    gids = lax.broadcasted_iota(jnp.int32, (_G, 1), 0)
    onehot = (batch_ref[...] == gids).astype(jnp.float32)
    summ = jnp.dot(onehot, hn, preferred_element_type=jnp.float32)
    cnt = jnp.sum(onehot, axis=1, keepdims=True)
    gx = summ / jnp.where(cnt > 0.0, cnt, 1.0)
    z = jnp.dot(gx, wm1_ref[...], preferred_element_type=jnp.float32) + bm1_ref[...]
    out_ref[...] = jnp.dot(z, wm2_ref[...], preferred_element_type=jnp.float32) + bm2_ref[...]


def _tc(body, out_shape, *args):
    return pl.pallas_call(
        body, out_shape=jax.ShapeDtypeStruct(out_shape, jnp.float32)
    )(*args)


@jax.jit
def kernel(x, edge_index, edge_attr, batch, W1, b1, W2, b2, W3, b3,
           gamma, beta, Wm1, bm1, Wm2, bm2):
    src = edge_index[0]
    dst = edge_index[1]
    ew = edge_attr.reshape(-1)

    dp = _deg_kernel(dst, ew)
    dp_t = dp.reshape(_NW, _N).T

    pad = _EPAD - _E
    src_p = jnp.concatenate([src, jnp.zeros((pad,), jnp.int32)])
    dst_p = jnp.concatenate([dst, jnp.zeros((pad,), jnp.int32)])
    ew_p = jnp.concatenate([ew, jnp.zeros((pad,), jnp.float32)])
    src_p = src_p.reshape(_NW, _NCHUNK, _CHUNK)
    dst_p = dst_p.reshape(_NW, _NCHUNK, _CHUNK)
    ew_p = ew_p.reshape(_NW, _NCHUNK, _CHUNK)

    dinv, hs1 = pl.pallas_call(
        _prep_body,
        out_shape=(
            jax.ShapeDtypeStruct((_N, 1), jnp.float32),
            jax.ShapeDtypeStruct((_N, _H), jnp.float32),
        ),
    )(dp_t, x, W1)

    p1 = _agg_kernel(src_p, dst_p, ew_p, hs1)
    hs2 = _tc(_mid_body, (_N, _H), p1, hs1, dinv, b1.reshape(1, _H), W2)
    p2 = _agg_kernel(src_p, dst_p, ew_p, hs2)
    hs3 = _tc(_mid_body, (_N, _H), p2, hs2, dinv, b2.reshape(1, _H), W3)
    p3 = _agg_kernel(src_p, dst_p, ew_p, hs3)

    pred = _tc(
        _head_body, (_G, _CLS),
        p3, hs3, dinv, b3.reshape(1, _H), gamma.reshape(1, _H),
        beta.reshape(1, _H), batch.reshape(1, _N), Wm1,
        bm1.reshape(1, _H), Wm2, bm2.reshape(1, _CLS),
    )
    return pred


# speedup vs baseline: 1.0350x; 1.0350x over previous
import functools
import jax
import jax.numpy as jnp
from jax import lax
from jax.experimental import pallas as pl
from jax.experimental.pallas import tpu as pltpu
from jax.experimental.pallas import tpu_sc as plsc

_N = 10000
_E = 320000
_CH = 128
_H = 64
_G = 128
_CLS = 10

_NC = 2
_NS = 16
_NW = _NC * _NS
_L = 16

_DEG_EPT = _E // _NW
_DEG_ROWS = _N // _L

_CHUNK = 128
_NCHUNK = 80
_EPT = _CHUNK * _NCHUNK
_EPAD = _EPT * _NW
_ZROWS = 125
_SROWS = _N // _NS

_mesh = plsc.VectorSubcoreMesh(core_axis_name="c", subcore_axis_name="s")


@functools.partial(
    pl.kernel,
    out_type=jax.ShapeDtypeStruct((_NW, _N), jnp.float32),
    mesh=_mesh,
    compiler_params=pltpu.CompilerParams(needs_layout_passes=False, use_tc_tiling_on_sc=False),
    scratch_types=[
        pltpu.VMEM((_DEG_EPT,), jnp.int32),
        pltpu.VMEM((_DEG_EPT,), jnp.float32),
        pltpu.VMEM((_N,), jnp.float32),
    ],
)
def _deg_kernel(dst_hbm, ew_hbm, out_hbm, dstv, ewv, degl):
    cid = lax.axis_index("c")
    sid = lax.axis_index("s")
    wid = sid * _NC + cid
    base = wid * _DEG_EPT
    pltpu.sync_copy(dst_hbm.at[pl.ds(base, _DEG_EPT)], dstv)
    pltpu.sync_copy(ew_hbm.at[pl.ds(base, _DEG_EPT)], ewv)

    def zero_body(i, carry):
        degl[pl.ds(i * _L, _L)] = jnp.zeros((_L,), jnp.float32)
        return carry

    lax.fori_loop(0, _N // _L, zero_body, 0)

    def edge_body(i, carry):
        d16 = dstv[pl.ds(i * _L, _L)]
        w16 = ewv[pl.ds(i * _L, _L)]
        plsc.addupdate_scatter(degl, [d16], w16)
        return carry

    lax.fori_loop(0, _DEG_EPT // _L, edge_body, 0)
    pltpu.sync_copy(degl, out_hbm.at[wid])


@functools.partial(
    pl.kernel,
    out_type=jax.ShapeDtypeStruct((_NC, _N, _H), jnp.float32),
    mesh=_mesh,
    compiler_params=pltpu.CompilerParams(needs_layout_passes=False, use_tc_tiling_on_sc=False),
    scratch_types=[
        pltpu.VMEM((_NCHUNK, _CHUNK), jnp.int32),
        pltpu.VMEM((_NCHUNK, _CHUNK), jnp.int32),
        pltpu.VMEM((_NCHUNK, _CHUNK), jnp.float32),
        pltpu.VMEM((_CHUNK, _H), jnp.float32),
        pltpu.VMEM((_CHUNK, _H), jnp.float32),
        pltpu.VMEM((_CHUNK, _H), jnp.float32),
        pltpu.VMEM((_CHUNK, _H), jnp.float32),
        pltpu.VMEM((_ZROWS, _H), jnp.float32),
        pltpu.VMEM_SHARED((_N, _H), jnp.float32),
        pltpu.SemaphoreType.DMA,
        pltpu.SemaphoreType.DMA,
        pltpu.SemaphoreType.DMA,
        pltpu.SemaphoreType.DMA,
        pltpu.SemaphoreType.DMA,
        pltpu.SemaphoreType.DMA,
        pltpu.SemaphoreType.DMA,
        pltpu.SemaphoreType.DMA,
    ],
)
def _agg_kernel(src_hbm, dst_hbm, ew_hbm, hs_hbm, out_hbm,
                src2d, dst2d, ew2d, rows0, rows1, rows2, rows3, zbuf, acc,
                sg0, sg1, sg2, sg3, ss0, ss1, ss2, ss3):
    cid = lax.axis_index("c")
    sid = lax.axis_index("s")
    wid = sid * _NC + cid
    pltpu.sync_copy(src_hbm.at[wid], src2d)
    pltpu.sync_copy(dst_hbm.at[wid], dst2d)
    pltpu.sync_copy(ew_hbm.at[wid], ew2d)

    def zrow_body(i, carry):
        for c4 in range(_H // _L):
            zbuf[i, pl.ds(c4 * _L, _L)] = jnp.zeros((_L,), jnp.float32)
        return carry

    lax.fori_loop(0, _ZROWS, zrow_body, 0)
    for k in range(_SROWS // _ZROWS):
        pltpu.sync_copy(zbuf, acc.at[pl.ds(sid * _SROWS + k * _ZROWS, _ZROWS)])
    plsc.subcore_barrier()

    def scale(rows, ci):
        def group_body(g, gcarry):
            ew16 = ew2d[ci, pl.ds(g * _L, _L)]
            for l in range(_L):
                s = ew16[l]
                e = g * _L + l
                for c4 in range(_H // _L):
                    rows[e, pl.ds(c4 * _L, _L)] = rows[e, pl.ds(c4 * _L, _L)] * s
            return gcarry

        lax.fori_loop(0, _CHUNK // _L, group_body, 0)

    bufs = (rows0, rows1, rows2, rows3)
    gsems = (sg0, sg1, sg2, sg3)
    ssems = (ss0, ss1, ss2, ss3)

    def quad_body(g, carry):
        c0 = 4 * g
        dgs = [
            pltpu.async_copy(hs_hbm.at[src2d.at[c0 + b]], bufs[b], gsems[b])
            for b in range(4)
        ]
        dss = []
        for b in range(4):
            dgs[b].wait()
            scale(bufs[b], c0 + b)
            dss.append(pltpu.async_copy(
                bufs[b], acc.at[dst2d.at[c0 + b]], ssems[b], add=True))
        for b in range(4):
            dss[b].wait()
        return carry

    lax.fori_loop(0, _NCHUNK // 4, quad_body, 0)
    plsc.subcore_barrier()
    pltpu.sync_copy(acc.at[pl.ds(sid * _SROWS, _SROWS)],
                    out_hbm.at[cid, pl.ds(sid * _SROWS, _SROWS)])


def _prep_body(dp_ref, x_ref, w1_ref, dinv_ref, hs_ref):
    deg = jnp.sum(dp_ref[...], axis=1, keepdims=True) + 1.0
    dinv = lax.rsqrt(deg)
    dinv_ref[...] = dinv
    xw = jnp.dot(x_ref[...], w1_ref[...], preferred_element_type=jnp.float32)
    hs_ref[...] = xw * dinv


def _mid_body(p_ref, hs_ref, dinv_ref, b_ref, w_ref, out_ref):
    dinv = dinv_ref[...]
    h = dinv * (p_ref[0] + p_ref[1] + hs_ref[...]) + b_ref[...]
    h = jnp.maximum(h, 0.0)
    out_ref[...] = jnp.dot(h, w_ref[...], preferred_element_type=jnp.float32) * dinv


def _head_body(p_ref, hs_ref, dinv_ref, b_ref, gamma_ref, beta_ref,
               batch_ref, wm1_ref, bm1_ref, wm2_ref, bm2_ref, out_ref):
    dinv = dinv_ref[...]
    h = dinv * (p_ref[0] + p_ref[1] + hs_ref[...]) + b_ref[...]
    h = jnp.maximum(h, 0.0)
    mean = jnp.mean(h, axis=0, keepdims=True)
    d = h - mean
    var = jnp.mean(d * d, axis=0, keepdims=True)
    hn = d * lax.rsqrt(var + 1e-5) * gamma_ref[...] + beta_ref[...]
    gids = lax.broadcasted_iota(jnp.int32, (_G, 1), 0)
    onehot = (batch_ref[...] == gids).astype(jnp.float32)
    summ = jnp.dot(onehot, hn, preferred_element_type=jnp.float32)
    cnt = jnp.sum(onehot, axis=1, keepdims=True)
    gx = summ / jnp.where(cnt > 0.0, cnt, 1.0)
    z = jnp.dot(gx, wm1_ref[...], preferred_element_type=jnp.float32) + bm1_ref[...]
    out_ref[...] = jnp.dot(z, wm2_ref[...], preferred_element_type=jnp.float32) + bm2_ref[...]


def _tc(body, out_shape, *args):
    return pl.pallas_call(
        body, out_shape=jax.ShapeDtypeStruct(out_shape, jnp.float32)
    )(*args)


@jax.jit
def kernel(x, edge_index, edge_attr, batch, W1, b1, W2, b2, W3, b3,
           gamma, beta, Wm1, bm1, Wm2, bm2):
    src = edge_index[0]
    dst = edge_index[1]
    ew = edge_attr.reshape(-1)

    dp = _deg_kernel(dst, ew)
    dp_t = dp.reshape(_NW, _N).T

    pad = _EPAD - _E
    src_p = jnp.concatenate([src, jnp.zeros((pad,), jnp.int32)])
    dst_p = jnp.concatenate([dst, jnp.zeros((pad,), jnp.int32)])
    ew_p = jnp.concatenate([ew, jnp.zeros((pad,), jnp.float32)])
    src_p = src_p.reshape(_NW, _NCHUNK, _CHUNK)
    dst_p = dst_p.reshape(_NW, _NCHUNK, _CHUNK)
    ew_p = ew_p.reshape(_NW, _NCHUNK, _CHUNK)

    dinv, hs1 = pl.pallas_call(
        _prep_body,
        out_shape=(
            jax.ShapeDtypeStruct((_N, 1), jnp.float32),
            jax.ShapeDtypeStruct((_N, _H), jnp.float32),
        ),
    )(dp_t, x, W1)

    p1 = _agg_kernel(src_p, dst_p, ew_p, hs1)
    hs2 = _tc(_mid_body, (_N, _H), p1, hs1, dinv, b1.reshape(1, _H), W2)
    p2 = _agg_kernel(src_p, dst_p, ew_p, hs2)
    hs3 = _tc(_mid_body, (_N, _H), p2, hs2, dinv, b2.reshape(1, _H), W3)
    p3 = _agg_kernel(src_p, dst_p, ew_p, hs3)

    pred = _tc(
        _head_body, (_G, _CLS),
        p3, hs3, dinv, b3.reshape(1, _H), gamma.reshape(1, _H),
        beta.reshape(1, _H), batch.reshape(1, _N), Wm1,
        bm1.reshape(1, _H), Wm2, bm2.reshape(1, _CLS),
    )
    return pred


# speedup vs baseline: 1.5463x; 1.4940x over previous
import functools
import jax
import jax.numpy as jnp
from jax import lax
from jax.experimental import pallas as pl
from jax.experimental.pallas import tpu as pltpu
from jax.experimental.pallas import tpu_sc as plsc

_N = 10000
_E = 320000
_CH = 128
_H = 64
_G = 128
_CLS = 10

_NC = 2
_NS = 16
_NW = _NC * _NS
_L = 16

_DEG_EPT = _E // _NW
_DEG_ROWS = _N // _L

_CHUNK = 128
_NCHUNK = 80
_EPT = _CHUNK * _NCHUNK
_EPAD = _EPT * _NW
_ZROWS = 125
_SROWS = _N // _NS

_mesh = plsc.VectorSubcoreMesh(core_axis_name="c", subcore_axis_name="s")


@functools.partial(
    pl.kernel,
    out_type=jax.ShapeDtypeStruct((_NW, _N), jnp.float32),
    mesh=_mesh,
    compiler_params=pltpu.CompilerParams(needs_layout_passes=False, use_tc_tiling_on_sc=False),
    scratch_types=[
        pltpu.VMEM((_DEG_EPT,), jnp.int32),
        pltpu.VMEM((_DEG_EPT,), jnp.float32),
        pltpu.VMEM((_N,), jnp.float32),
    ],
)
def _deg_kernel(dst_hbm, ew_hbm, out_hbm, dstv, ewv, degl):
    cid = lax.axis_index("c")
    sid = lax.axis_index("s")
    wid = sid * _NC + cid
    base = wid * _DEG_EPT
    pltpu.sync_copy(dst_hbm.at[pl.ds(base, _DEG_EPT)], dstv)
    pltpu.sync_copy(ew_hbm.at[pl.ds(base, _DEG_EPT)], ewv)

    def zero_body(i, carry):
        degl[pl.ds(i * _L, _L)] = jnp.zeros((_L,), jnp.float32)
        return carry

    lax.fori_loop(0, _N // _L, zero_body, 0)

    def edge_body(i, carry):
        d16 = dstv[pl.ds(i * _L, _L)]
        w16 = ewv[pl.ds(i * _L, _L)]
        plsc.addupdate_scatter(degl, [d16], w16)
        return carry

    lax.fori_loop(0, _DEG_EPT // _L, edge_body, 0)
    pltpu.sync_copy(degl, out_hbm.at[wid])


@functools.partial(
    pl.kernel,
    out_type=jax.ShapeDtypeStruct((_NC, _N, _H), jnp.float32),
    mesh=_mesh,
    compiler_params=pltpu.CompilerParams(needs_layout_passes=False, use_tc_tiling_on_sc=False),
    scratch_types=[
        pltpu.VMEM((_NCHUNK, _CHUNK), jnp.int32),
        pltpu.VMEM((_NCHUNK, _CHUNK), jnp.int32),
        pltpu.VMEM((_NCHUNK, _CHUNK), jnp.float32),
        pltpu.VMEM((_CHUNK, _H), jnp.float32),
        pltpu.VMEM((_CHUNK, _H), jnp.float32),
        pltpu.VMEM((_CHUNK, _H), jnp.float32),
        pltpu.VMEM((_CHUNK, _H), jnp.float32),
        pltpu.VMEM((_ZROWS, _H), jnp.float32),
        pltpu.VMEM_SHARED((_N, _H), jnp.float32),
        pltpu.VMEM_SHARED((_N, _H), jnp.float32),
        pltpu.SemaphoreType.DMA,
        pltpu.SemaphoreType.DMA,
    ],
)
def _agg_kernel(src_hbm, dst_hbm, ew_hbm, hs_hbm, out_hbm,
                src2d, dst2d, ew2d, rows0, rows1, rows2, rows3, zbuf, acc,
                hs_spm, sg0, sg1):
    cid = lax.axis_index("c")
    sid = lax.axis_index("s")
    wid = sid * _NC + cid
    pltpu.sync_copy(src_hbm.at[wid], src2d)
    pltpu.sync_copy(dst_hbm.at[wid], dst2d)
    pltpu.sync_copy(ew_hbm.at[wid], ew2d)

    def zrow_body(i, carry):
        for c4 in range(_H // _L):
            zbuf[i, pl.ds(c4 * _L, _L)] = jnp.zeros((_L,), jnp.float32)
        return carry

    lax.fori_loop(0, _ZROWS, zrow_body, 0)
    for k in range(_SROWS // _ZROWS):
        pltpu.sync_copy(zbuf, acc.at[pl.ds(sid * _SROWS + k * _ZROWS, _ZROWS)])
    pltpu.sync_copy(hs_hbm.at[pl.ds(sid * _SROWS, _SROWS)],
                    hs_spm.at[pl.ds(sid * _SROWS, _SROWS)])
    plsc.subcore_barrier()

    def scale(rows, ci):
        def group_body(g, gcarry):
            ew16 = ew2d[ci, pl.ds(g * _L, _L)]
            for l in range(_L):
                s = ew16[l]
                e = g * _L + l
                for c4 in range(_H // _L):
                    rows[e, pl.ds(c4 * _L, _L)] = rows[e, pl.ds(c4 * _L, _L)] * s
            return gcarry

        lax.fori_loop(0, _CHUNK // _L, group_body, 0)

    def chunk_body(ci, carry):
        pltpu.async_copy(hs_spm.at[src2d.at[ci]], rows0, sg0).wait()
        scale(rows0, ci)
        pltpu.sync_copy(rows0, acc.at[dst2d.at[ci]], add=True)
        return carry

    lax.fori_loop(0, _NCHUNK, chunk_body, 0)
    plsc.subcore_barrier()
    pltpu.sync_copy(acc.at[pl.ds(sid * _SROWS, _SROWS)],
                    out_hbm.at[cid, pl.ds(sid * _SROWS, _SROWS)])


def _prep_body(dp_ref, x_ref, w1_ref, dinv_ref, hs_ref):
    deg = jnp.sum(dp_ref[...], axis=1, keepdims=True) + 1.0
    dinv = lax.rsqrt(deg)
    dinv_ref[...] = dinv
    xw = jnp.dot(x_ref[...], w1_ref[...], preferred_element_type=jnp.float32)
    hs_ref[...] = xw * dinv


def _mid_body(p_ref, hs_ref, dinv_ref, b_ref, w_ref, out_ref):
    dinv = dinv_ref[...]
    h = dinv * (p_ref[0] + p_ref[1] + hs_ref[...]) + b_ref[...]
    h = jnp.maximum(h, 0.0)
    out_ref[...] = jnp.dot(h, w_ref[...], preferred_element_type=jnp.float32) * dinv


def _head_body(p_ref, hs_ref, dinv_ref, b_ref, gamma_ref, beta_ref,
               batch_ref, wm1_ref, bm1_ref, wm2_ref, bm2_ref, out_ref):
    dinv = dinv_ref[...]
    h = dinv * (p_ref[0] + p_ref[1] + hs_ref[...]) + b_ref[...]
    h = jnp.maximum(h, 0.0)
    mean = jnp.mean(h, axis=0, keepdims=True)
    d = h - mean
    var = jnp.mean(d * d, axis=0, keepdims=True)
    hn = d * lax.rsqrt(var + 1e-5) * gamma_ref[...] + beta_ref[...]
    gids = lax.broadcasted_iota(jnp.int32, (_G, 1), 0)
    onehot = (batch_ref[...] == gids).astype(jnp.float32)
    summ = jnp.dot(onehot, hn, preferred_element_type=jnp.float32)
    cnt = jnp.sum(onehot, axis=1, keepdims=True)
    gx = summ / jnp.where(cnt > 0.0, cnt, 1.0)
    z = jnp.dot(gx, wm1_ref[...], preferred_element_type=jnp.float32) + bm1_ref[...]
    out_ref[...] = jnp.dot(z, wm2_ref[...], preferred_element_type=jnp.float32) + bm2_ref[...]


def _tc(body, out_shape, *args):
    return pl.pallas_call(
        body, out_shape=jax.ShapeDtypeStruct(out_shape, jnp.float32)
    )(*args)


@jax.jit
def kernel(x, edge_index, edge_attr, batch, W1, b1, W2, b2, W3, b3,
           gamma, beta, Wm1, bm1, Wm2, bm2):
    src = edge_index[0]
    dst = edge_index[1]
    ew = edge_attr.reshape(-1)

    dp = _deg_kernel(dst, ew)
    dp_t = dp.reshape(_NW, _N).T

    pad = _EPAD - _E
    src_p = jnp.concatenate([src, jnp.zeros((pad,), jnp.int32)])
    dst_p = jnp.concatenate([dst, jnp.zeros((pad,), jnp.int32)])
    ew_p = jnp.concatenate([ew, jnp.zeros((pad,), jnp.float32)])
    src_p = src_p.reshape(_NW, _NCHUNK, _CHUNK)
    dst_p = dst_p.reshape(_NW, _NCHUNK, _CHUNK)
    ew_p = ew_p.reshape(_NW, _NCHUNK, _CHUNK)

    dinv, hs1 = pl.pallas_call(
        _prep_body,
        out_shape=(
            jax.ShapeDtypeStruct((_N, 1), jnp.float32),
            jax.ShapeDtypeStruct((_N, _H), jnp.float32),
        ),
    )(dp_t, x, W1)

    p1 = _agg_kernel(src_p, dst_p, ew_p, hs1)
    hs2 = _tc(_mid_body, (_N, _H), p1, hs1, dinv, b1.reshape(1, _H), W2)
    p2 = _agg_kernel(src_p, dst_p, ew_p, hs2)
    hs3 = _tc(_mid_body, (_N, _H), p2, hs2, dinv, b2.reshape(1, _H), W3)
    p3 = _agg_kernel(src_p, dst_p, ew_p, hs3)

    pred = _tc(
        _head_body, (_G, _CLS),
        p3, hs3, dinv, b3.reshape(1, _H), gamma.reshape(1, _H),
        beta.reshape(1, _H), batch.reshape(1, _N), Wm1,
        bm1.reshape(1, _H), Wm2, bm2.reshape(1, _CLS),
    )
    return pred


# speedup vs baseline: 2.4446x; 1.5809x over previous
import functools
import jax
import jax.numpy as jnp
from jax import lax
from jax.experimental import pallas as pl
from jax.experimental.pallas import tpu as pltpu
from jax.experimental.pallas import tpu_sc as plsc

_N = 10000
_E = 320000
_CH = 128
_H = 64
_G = 128
_CLS = 10

_NC = 2
_NS = 16
_NW = _NC * _NS
_L = 16

_DEG_EPT = _E // _NW
_DEG_ROWS = _N // _L

_CHUNK = 128
_NCHUNK = 160
_EPT = _CHUNK * _NCHUNK
_EPAD = _EPT * _NS
_HH = _H // _NC
_ZROWS = 125
_SROWS = _N // _NS

_mesh = plsc.VectorSubcoreMesh(core_axis_name="c", subcore_axis_name="s")


@functools.partial(
    pl.kernel,
    out_type=jax.ShapeDtypeStruct((_NW, _N), jnp.float32),
    mesh=_mesh,
    compiler_params=pltpu.CompilerParams(needs_layout_passes=False, use_tc_tiling_on_sc=False),
    scratch_types=[
        pltpu.VMEM((_DEG_EPT,), jnp.int32),
        pltpu.VMEM((_DEG_EPT,), jnp.float32),
        pltpu.VMEM((_N,), jnp.float32),
    ],
)
def _deg_kernel(dst_hbm, ew_hbm, out_hbm, dstv, ewv, degl):
    cid = lax.axis_index("c")
    sid = lax.axis_index("s")
    wid = sid * _NC + cid
    base = wid * _DEG_EPT
    pltpu.sync_copy(dst_hbm.at[pl.ds(base, _DEG_EPT)], dstv)
    pltpu.sync_copy(ew_hbm.at[pl.ds(base, _DEG_EPT)], ewv)

    def zero_body(i, carry):
        degl[pl.ds(i * _L, _L)] = jnp.zeros((_L,), jnp.float32)
        return carry

    lax.fori_loop(0, _N // _L, zero_body, 0)

    def edge_body(i, carry):
        d16 = dstv[pl.ds(i * _L, _L)]
        w16 = ewv[pl.ds(i * _L, _L)]
        plsc.addupdate_scatter(degl, [d16], w16)
        return carry

    lax.fori_loop(0, _DEG_EPT // _L, edge_body, 0)
    pltpu.sync_copy(degl, out_hbm.at[wid])


@functools.partial(
    pl.kernel,
    out_type=jax.ShapeDtypeStruct((_NC, _N, _HH), jnp.float32),
    mesh=_mesh,
    compiler_params=pltpu.CompilerParams(needs_layout_passes=False, use_tc_tiling_on_sc=False),
    scratch_types=[
        pltpu.VMEM((_NCHUNK, _CHUNK), jnp.int32),
        pltpu.VMEM((_NCHUNK, _CHUNK), jnp.int32),
        pltpu.VMEM((_NCHUNK, _CHUNK), jnp.float32),
        pltpu.VMEM((_CHUNK, _HH), jnp.float32),
        pltpu.VMEM((_CHUNK, _HH), jnp.float32),
        pltpu.VMEM((_ZROWS, _HH), jnp.float32),
        pltpu.VMEM_SHARED((_N, _HH), jnp.float32),
        pltpu.VMEM_SHARED((_N, _HH), jnp.float32),
        pltpu.SemaphoreType.DMA,
        pltpu.SemaphoreType.DMA,
    ],
)
def _agg_kernel(src_hbm, dst_hbm, ew_hbm, hs_hbm, out_hbm,
                src2d, dst2d, ew2d, rows0, rows1, zbuf, acc,
                hs_spm, sg0, sg1):
    cid = lax.axis_index("c")
    sid = lax.axis_index("s")
    pltpu.sync_copy(src_hbm.at[sid], src2d)
    pltpu.sync_copy(dst_hbm.at[sid], dst2d)
    pltpu.sync_copy(ew_hbm.at[sid], ew2d)

    def zrow_body(i, carry):
        for c4 in range(_HH // _L):
            zbuf[i, pl.ds(c4 * _L, _L)] = jnp.zeros((_L,), jnp.float32)
        return carry

    lax.fori_loop(0, _ZROWS, zrow_body, 0)
    for k in range(_SROWS // _ZROWS):
        pltpu.sync_copy(zbuf, acc.at[pl.ds(sid * _SROWS + k * _ZROWS, _ZROWS)])
    pltpu.sync_copy(hs_hbm.at[cid, pl.ds(sid * _SROWS, _SROWS)],
                    hs_spm.at[pl.ds(sid * _SROWS, _SROWS)])
    plsc.subcore_barrier()

    def scale(rows, ci):
        def group_body(g, gcarry):
            ew16 = ew2d[ci, pl.ds(g * _L, _L)]
            for l in range(_L):
                s = ew16[l]
                e = g * _L + l
                for c4 in range(_HH // _L):
                    rows[e, pl.ds(c4 * _L, _L)] = rows[e, pl.ds(c4 * _L, _L)] * s
            return gcarry

        lax.fori_loop(0, _CHUNK // _L, group_body, 0)

    def pair_body(g, carry):
        c0 = 2 * g
        c1 = 2 * g + 1
        dg0 = pltpu.async_copy(hs_spm.at[src2d.at[c0]], rows0, sg0)
        dg1 = pltpu.async_copy(hs_spm.at[src2d.at[c1]], rows1, sg1)
        dg0.wait()
        scale(rows0, c0)
        pltpu.sync_copy(rows0, acc.at[dst2d.at[c0]], add=True)
        dg1.wait()
        scale(rows1, c1)
        pltpu.sync_copy(rows1, acc.at[dst2d.at[c1]], add=True)
        return carry

    lax.fori_loop(0, _NCHUNK // 2, pair_body, 0)
    plsc.subcore_barrier()
    pltpu.sync_copy(acc.at[pl.ds(sid * _SROWS, _SROWS)],
                    out_hbm.at[cid, pl.ds(sid * _SROWS, _SROWS)])


def _prep_body(dp_ref, x_ref, w1_ref, dinv_ref, hs_ref):
    deg = jnp.sum(dp_ref[...], axis=1, keepdims=True) + 1.0
    dinv = lax.rsqrt(deg)
    dinv_ref[...] = dinv
    xw = jnp.dot(x_ref[...], w1_ref[...], preferred_element_type=jnp.float32)
    hs = xw * dinv
    hs_ref[0] = hs[:, :_HH]
    hs_ref[1] = hs[:, _HH:]


def _mid_body(p_ref, hs_ref, dinv_ref, b_ref, w_ref, out_ref):
    dinv = dinv_ref[...]
    s = jnp.concatenate([p_ref[0] + hs_ref[0], p_ref[1] + hs_ref[1]], axis=1)
    h = dinv * s + b_ref[...]
    h = jnp.maximum(h, 0.0)
    hs = jnp.dot(h, w_ref[...], preferred_element_type=jnp.float32) * dinv
    out_ref[0] = hs[:, :_HH]
    out_ref[1] = hs[:, _HH:]


def _head_body(p_ref, hs_ref, dinv_ref, b_ref, gamma_ref, beta_ref,
               batch_ref, wm1_ref, bm1_ref, wm2_ref, bm2_ref, out_ref):
    dinv = dinv_ref[...]
    s = jnp.concatenate([p_ref[0] + hs_ref[0], p_ref[1] + hs_ref[1]], axis=1)
    h = dinv * s + b_ref[...]
    h = jnp.maximum(h, 0.0)
    mean = jnp.mean(h, axis=0, keepdims=True)
    d = h - mean
    var = jnp.mean(d * d, axis=0, keepdims=True)
    hn = d * lax.rsqrt(var + 1e-5) * gamma_ref[...] + beta_ref[...]
    gids = lax.broadcasted_iota(jnp.int32, (_G, 1), 0)
    onehot = (batch_ref[...] == gids).astype(jnp.float32)
    summ = jnp.dot(onehot, hn, preferred_element_type=jnp.float32)
    cnt = jnp.sum(onehot, axis=1, keepdims=True)
    gx = summ / jnp.where(cnt > 0.0, cnt, 1.0)
    z = jnp.dot(gx, wm1_ref[...], preferred_element_type=jnp.float32) + bm1_ref[...]
    out_ref[...] = jnp.dot(z, wm2_ref[...], preferred_element_type=jnp.float32) + bm2_ref[...]


@jax.jit
def kernel(x, edge_index, edge_attr, batch, W1, b1, W2, b2, W3, b3,
           gamma, beta, Wm1, bm1, Wm2, bm2):
    src = edge_index[0]
    dst = edge_index[1]
    ew = edge_attr.reshape(-1)

    dp = _deg_kernel(dst, ew)
    dp_t = dp.reshape(_NW, _N).T

    pad = _EPAD - _E
    src_p = jnp.concatenate([src, jnp.zeros((pad,), jnp.int32)])
    dst_p = jnp.concatenate([dst, jnp.zeros((pad,), jnp.int32)])
    ew_p = jnp.concatenate([ew, jnp.zeros((pad,), jnp.float32)])
    src_p = src_p.reshape(_NS, _NCHUNK, _CHUNK)
    dst_p = dst_p.reshape(_NS, _NCHUNK, _CHUNK)
    ew_p = ew_p.reshape(_NS, _NCHUNK, _CHUNK)

    hs_shape = (
        jax.ShapeDtypeStruct((_N, 1), jnp.float32),
        jax.ShapeDtypeStruct((_NC, _N, _HH), jnp.float32),
    )
    dinv, hs1 = pl.pallas_call(_prep_body, out_shape=hs_shape)(dp_t, x, W1)

    def _mid(p, hs, b, W):
        return pl.pallas_call(
            _mid_body,
            out_shape=jax.ShapeDtypeStruct((_NC, _N, _HH), jnp.float32),
        )(p, hs, dinv, b.reshape(1, _H), W)

    p1 = _agg_kernel(src_p, dst_p, ew_p, hs1)
    hs2 = _mid(p1, hs1, b1, W2)
    p2 = _agg_kernel(src_p, dst_p, ew_p, hs2)
    hs3 = _mid(p2, hs2, b2, W3)
    p3 = _agg_kernel(src_p, dst_p, ew_p, hs3)

    pred = pl.pallas_call(
        _head_body, out_shape=jax.ShapeDtypeStruct((_G, _CLS), jnp.float32)
    )(
        p3, hs3, dinv, b3.reshape(1, _H), gamma.reshape(1, _H),
        beta.reshape(1, _H), batch.reshape(1, _N), Wm1,
        bm1.reshape(1, _H), Wm2, bm2.reshape(1, _CLS),
    )
    return pred


# speedup vs baseline: 2.7097x; 1.1084x over previous
import functools
import jax
import jax.numpy as jnp
from jax import lax
from jax.experimental import pallas as pl
from jax.experimental.pallas import tpu as pltpu
from jax.experimental.pallas import tpu_sc as plsc

_N = 10000
_E = 320000
_CH = 128
_H = 64
_G = 128
_CLS = 10

_NC = 2
_NS = 16
_NW = _NC * _NS
_L = 16

_DEG_EPT = _E // _NW
_DEG_ROWS = _N // _L

_CHUNK = 128
_NCHUNK = 160
_EPT = _CHUNK * _NCHUNK
_EPAD = _EPT * _NS
_HH = _H // _NC
_ZROWS = 125
_SROWS = _N // _NS

_mesh = plsc.VectorSubcoreMesh(core_axis_name="c", subcore_axis_name="s")


@functools.partial(
    pl.kernel,
    out_type=jax.ShapeDtypeStruct((_NW, _N), jnp.float32),
    mesh=_mesh,
    compiler_params=pltpu.CompilerParams(needs_layout_passes=False, use_tc_tiling_on_sc=False),
    scratch_types=[
        pltpu.VMEM((_DEG_EPT,), jnp.int32),
        pltpu.VMEM((_DEG_EPT,), jnp.float32),
        pltpu.VMEM((_N,), jnp.float32),
    ],
)
def _deg_kernel(dst_hbm, ew_hbm, out_hbm, dstv, ewv, degl):
    cid = lax.axis_index("c")
    sid = lax.axis_index("s")
    wid = sid * _NC + cid
    base = wid * _DEG_EPT
    pltpu.sync_copy(dst_hbm.at[pl.ds(base, _DEG_EPT)], dstv)
    pltpu.sync_copy(ew_hbm.at[pl.ds(base, _DEG_EPT)], ewv)

    def zero_body(i, carry):
        degl[pl.ds(i * _L, _L)] = jnp.zeros((_L,), jnp.float32)
        return carry

    lax.fori_loop(0, _N // _L, zero_body, 0)

    def edge_body(i, carry):
        d16 = dstv[pl.ds(i * _L, _L)]
        w16 = ewv[pl.ds(i * _L, _L)]
        plsc.addupdate_scatter(degl, [d16], w16)
        return carry

    lax.fori_loop(0, _DEG_EPT // _L, edge_body, 0)
    pltpu.sync_copy(degl, out_hbm.at[wid])


@functools.partial(
    pl.kernel,
    out_type=jax.ShapeDtypeStruct((_NC, _N, _HH), jnp.float32),
    mesh=_mesh,
    compiler_params=pltpu.CompilerParams(needs_layout_passes=False, use_tc_tiling_on_sc=False),
    scratch_types=[
        pltpu.VMEM((_NCHUNK, _CHUNK), jnp.int32),
        pltpu.VMEM((_NCHUNK, _CHUNK), jnp.int32),
        pltpu.VMEM((_NCHUNK, _CHUNK), jnp.float32),
        pltpu.VMEM((_CHUNK, _HH), jnp.float32),
        pltpu.VMEM((_CHUNK, _HH), jnp.float32),
        pltpu.VMEM((_ZROWS, _HH), jnp.float32),
        pltpu.VMEM_SHARED((_N, _HH), jnp.float32),
        pltpu.VMEM_SHARED((_N, _HH), jnp.float32),
        pltpu.SemaphoreType.DMA,
        pltpu.SemaphoreType.DMA,
        pltpu.SemaphoreType.DMA,
        pltpu.SemaphoreType.DMA,
    ],
)
def _agg_kernel(src_hbm, dst_hbm, ew_hbm, hs_hbm, out_hbm,
                src2d, dst2d, ew2d, rows0, rows1, zbuf, acc,
                hs_spm, sg0, sg1, ss0, ss1):
    cid = lax.axis_index("c")
    sid = lax.axis_index("s")
    pltpu.sync_copy(src_hbm.at[sid], src2d)
    pltpu.sync_copy(dst_hbm.at[sid], dst2d)
    pltpu.sync_copy(ew_hbm.at[sid], ew2d)

    def zrow_body(i, carry):
        for c4 in range(_HH // _L):
            zbuf[i, pl.ds(c4 * _L, _L)] = jnp.zeros((_L,), jnp.float32)
        return carry

    lax.fori_loop(0, _ZROWS, zrow_body, 0)
    for k in range(_SROWS // _ZROWS):
        pltpu.sync_copy(zbuf, acc.at[pl.ds(sid * _SROWS + k * _ZROWS, _ZROWS)])
    pltpu.sync_copy(hs_hbm.at[cid, pl.ds(sid * _SROWS, _SROWS)],
                    hs_spm.at[pl.ds(sid * _SROWS, _SROWS)])
    plsc.subcore_barrier()

    def scale(rows, ci):
        def group_body(g, gcarry):
            ew16 = ew2d[ci, pl.ds(g * _L, _L)]
            for l in range(_L):
                s = ew16[l]
                e = g * _L + l
                for c4 in range(_HH // _L):
                    rows[e, pl.ds(c4 * _L, _L)] = rows[e, pl.ds(c4 * _L, _L)] * s
            return gcarry

        lax.fori_loop(0, _CHUNK // _L, group_body, 0)

    def quad_body(g, carry):
        c0 = 4 * g
        dg0 = pltpu.async_copy(hs_spm.at[src2d.at[c0]], rows0, sg0)
        dg1 = pltpu.async_copy(hs_spm.at[src2d.at[c0 + 1]], rows1, sg1)
        dg0.wait()
        scale(rows0, c0)
        ds0 = pltpu.async_copy(rows0, acc.at[dst2d.at[c0]], ss0, add=True)
        dg1.wait()
        scale(rows1, c0 + 1)
        ds1 = pltpu.async_copy(rows1, acc.at[dst2d.at[c0 + 1]], ss1, add=True)
        ds0.wait()
        dg2 = pltpu.async_copy(hs_spm.at[src2d.at[c0 + 2]], rows0, sg0)
        ds1.wait()
        dg3 = pltpu.async_copy(hs_spm.at[src2d.at[c0 + 3]], rows1, sg1)
        dg2.wait()
        scale(rows0, c0 + 2)
        ds2 = pltpu.async_copy(rows0, acc.at[dst2d.at[c0 + 2]], ss0, add=True)
        dg3.wait()
        scale(rows1, c0 + 3)
        ds3 = pltpu.async_copy(rows1, acc.at[dst2d.at[c0 + 3]], ss1, add=True)
        ds2.wait()
        ds3.wait()
        return carry

    lax.fori_loop(0, _NCHUNK // 4, quad_body, 0)
    plsc.subcore_barrier()
    pltpu.sync_copy(acc.at[pl.ds(sid * _SROWS, _SROWS)],
                    out_hbm.at[cid, pl.ds(sid * _SROWS, _SROWS)])


def _prep_body(dp_ref, x_ref, w1_ref, dinv_ref, hs_ref):
    deg = jnp.sum(dp_ref[...], axis=1, keepdims=True) + 1.0
    dinv = lax.rsqrt(deg)
    dinv_ref[...] = dinv
    xw = jnp.dot(x_ref[...], w1_ref[...], preferred_element_type=jnp.float32)
    hs = xw * dinv
    hs_ref[0] = hs[:, :_HH]
    hs_ref[1] = hs[:, _HH:]


def _mid_body(p_ref, hs_ref, dinv_ref, b_ref, w_ref, out_ref):
    dinv = dinv_ref[...]
    s = jnp.concatenate([p_ref[0] + hs_ref[0], p_ref[1] + hs_ref[1]], axis=1)
    h = dinv * s + b_ref[...]
    h = jnp.maximum(h, 0.0)
    hs = jnp.dot(h, w_ref[...], preferred_element_type=jnp.float32) * dinv
    out_ref[0] = hs[:, :_HH]
    out_ref[1] = hs[:, _HH:]


def _head_body(p_ref, hs_ref, dinv_ref, b_ref, gamma_ref, beta_ref,
               batch_ref, wm1_ref, bm1_ref, wm2_ref, bm2_ref, out_ref):
    dinv = dinv_ref[...]
    s = jnp.concatenate([p_ref[0] + hs_ref[0], p_ref[1] + hs_ref[1]], axis=1)
    h = dinv * s + b_ref[...]
    h = jnp.maximum(h, 0.0)
    mean = jnp.mean(h, axis=0, keepdims=True)
    d = h - mean
    var = jnp.mean(d * d, axis=0, keepdims=True)
    hn = d * lax.rsqrt(var + 1e-5) * gamma_ref[...] + beta_ref[...]
    gids = lax.broadcasted_iota(jnp.int32, (_G, 1), 0)
    onehot = (batch_ref[...] == gids).astype(jnp.float32)
    summ = jnp.dot(onehot, hn, preferred_element_type=jnp.float32)
    cnt = jnp.sum(onehot, axis=1, keepdims=True)
    gx = summ / jnp.where(cnt > 0.0, cnt, 1.0)
    z = jnp.dot(gx, wm1_ref[...], preferred_element_type=jnp.float32) + bm1_ref[...]
    out_ref[...] = jnp.dot(z, wm2_ref[...], preferred_element_type=jnp.float32) + bm2_ref[...]


@jax.jit
def kernel(x, edge_index, edge_attr, batch, W1, b1, W2, b2, W3, b3,
           gamma, beta, Wm1, bm1, Wm2, bm2):
    src = edge_index[0]
    dst = edge_index[1]
    ew = edge_attr.reshape(-1)

    dp = _deg_kernel(dst, ew)
    dp_t = dp.reshape(_NW, _N).T

    pad = _EPAD - _E
    src_p = jnp.concatenate([src, jnp.zeros((pad,), jnp.int32)])
    dst_p = jnp.concatenate([dst, jnp.zeros((pad,), jnp.int32)])
    ew_p = jnp.concatenate([ew, jnp.zeros((pad,), jnp.float32)])
    src_p = src_p.reshape(_NS, _NCHUNK, _CHUNK)
    dst_p = dst_p.reshape(_NS, _NCHUNK, _CHUNK)
    ew_p = ew_p.reshape(_NS, _NCHUNK, _CHUNK)

    hs_shape = (
        jax.ShapeDtypeStruct((_N, 1), jnp.float32),
        jax.ShapeDtypeStruct((_NC, _N, _HH), jnp.float32),
    )
    dinv, hs1 = pl.pallas_call(_prep_body, out_shape=hs_shape)(dp_t, x, W1)

    def _mid(p, hs, b, W):
        return pl.pallas_call(
            _mid_body,
            out_shape=jax.ShapeDtypeStruct((_NC, _N, _HH), jnp.float32),
        )(p, hs, dinv, b.reshape(1, _H), W)

    p1 = _agg_kernel(src_p, dst_p, ew_p, hs1)
    hs2 = _mid(p1, hs1, b1, W2)
    p2 = _agg_kernel(src_p, dst_p, ew_p, hs2)
    hs3 = _mid(p2, hs2, b2, W3)
    p3 = _agg_kernel(src_p, dst_p, ew_p, hs3)

    pred = pl.pallas_call(
        _head_body, out_shape=jax.ShapeDtypeStruct((_G, _CLS), jnp.float32)
    )(
        p3, hs3, dinv, b3.reshape(1, _H), gamma.reshape(1, _H),
        beta.reshape(1, _H), batch.reshape(1, _N), Wm1,
        bm1.reshape(1, _H), Wm2, bm2.reshape(1, _CLS),
    )
    return pred
